# Initial kernel scaffold; baseline (speedup 1.0000x reference)
import jax
import jax.numpy as jnp
from jax.experimental import pallas as pl


def kernel(bond_index, bond_feat_cate, bond_feat_float, bond_mask, structure_feat_cate, structure_feat_float, bond_cate_table, bond_cate_ln_g, bond_cate_ln_b, bond_float_W, bond_float_b, bond_float_ln_g, bond_float_ln_b, struct_cate_table, struct_cate_ln_g, struct_cate_ln_b, struct_float_W, struct_float_b, struct_float_ln_g, struct_float_ln_b, virtual_edge_emb, rev_W, rev_b):
    raise NotImplementedError("write your pallas kernel here")


# speedup vs baseline: 4.3024x; 4.3024x over previous
import jax
import jax.numpy as jnp
from jax import lax
from jax.experimental import pallas as pl
from jax.experimental.pallas import tpu as pltpu
from jax.experimental.pallas import tpu_sc as plsc

B, E, N, H = 16, 256, 127, 64
MAXN = N + 1
CELLS = MAXN * MAXN
PROWS = CELLS // 2
NUPD = 2 * E
F32 = jnp.float32
HIGH = jax.lax.Precision.HIGHEST


def _ln(x, g, b, eps=1e-5):
    m = jnp.mean(x, axis=-1, keepdims=True)
    d = x - m
    v = jnp.mean(d * d, axis=-1, keepdims=True)
    return d * jax.lax.rsqrt(v + eps) * g + b


def _struct_half(cate, ff, sct, scg, scb, sfW, sfb, sfg, sfb2):
    p = cate.shape[0]
    idx0 = cate[:, 0:1] + 1
    idx1 = cate[:, 1:2] + 34
    t = lax.broadcasted_iota(jnp.int32, (p, 50), 1)
    oh = (t == idx0).astype(F32) + (t == idx1).astype(F32)
    ce = jnp.dot(oh, sct, precision=HIGH, preferred_element_type=F32)
    ce = _ln(ce, scg, scb)
    fe = jnp.dot(ff, sfW, precision=HIGH, preferred_element_type=F32) + sfb
    return ce + _ln(fe, sfg, sfb2)


def _dense_body(
    sfc_e_ref, sfc_o_ref, sff_e_ref, sff_o_ref,
    bfc_ref, bff_ref, bmask_ref,
    bidx_a_ref, bidx_col_ref, bidx_row_ref,
    bct_ref, bcg_ref, bcb_ref,
    bfW_ref, bfb_ref, bfg_ref, bfb2_ref,
    sct_ref, scg_ref, scb_ref,
    sfW_ref, sfb_ref, sfg_ref, sfb2_ref,
    ve_ref, revW_ref, revb_ref,
    hs_ref, v_ref, c_ref,
):
    bprog = pl.program_id(0)
    sct, scg, scb = sct_ref[...], scg_ref[...], scb_ref[...]
    sfW, sfb = sfW_ref[...], sfb_ref[...]
    sfg, sfb2 = sfg_ref[...], sfb2_ref[...]
    ve = ve_ref[...]

    chunk = 2048
    for ch in range(PROWS // chunk):
        sl = pl.ds(ch * chunk, chunk)
        val_e = _struct_half(sfc_e_ref[0, sl, :], sff_e_ref[0, sl, :],
                             sct, scg, scb, sfW, sfb, sfg, sfb2)
        val_o = _struct_half(sfc_o_ref[0, sl, :], sff_o_ref[0, sl, :],
                             sct, scg, scb, sfW, sfb, sfg, sfb2)
        q = lax.broadcasted_iota(jnp.int32, (chunk, 1), 0) + ch * chunk
        isve_e = jnp.logical_or(q < MAXN // 2, q % (MAXN // 2) == 0)
        isve_o = q < MAXN // 2
        val_e = jnp.where(isve_e, ve, val_e)
        val_o = jnp.where(isve_o, ve, val_o)
        hs_ref[0, sl, :] = jnp.concatenate([val_e, val_o], axis=1)

    bc = bfc_ref[0]
    i0 = bc[:, 0:1] + 1
    i1 = bc[:, 1:2] + 18
    i2 = bc[:, 2:3] + 27
    t = lax.broadcasted_iota(jnp.int32, (E, 35), 1)
    oh = ((t == i0).astype(F32) + (t == i1).astype(F32)
          + (t == i2).astype(F32))
    ce = jnp.dot(oh, bct_ref[...], precision=HIGH,
                 preferred_element_type=F32)
    ce = _ln(ce, bcg_ref[...], bcb_ref[...])
    fe = jnp.dot(bff_ref[0], bfW_ref[...], precision=HIGH,
                 preferred_element_type=F32) + bfb_ref[...]
    fe = _ln(fe, bfg_ref[...], bfb2_ref[...])
    hb = (ce + fe) * bmask_ref[0]
    rev = jnp.dot(hb, revW_ref[...], precision=HIGH,
                  preferred_element_type=F32) + revb_ref[...]
    upd = jnp.concatenate([hb, rev], axis=0)

    bi = bidx_a_ref[0] + 1
    bi0 = bi[0]
    bi1 = bi[1]
    qf = bi0 * (MAXN // 2) + (bi1 // 2)
    qr = bi1 * (MAXN // 2) + (bi0 // 2)
    c_ref[0] = jnp.concatenate([qf, qr], axis=0) + bprog * PROWS

    col = bidx_col_ref[0] + 1
    row = bidx_row_ref[0] + 1
    pk = col[:, 0:1] * MAXN + col[:, 1:2]
    plr = row[0:1, :] * MAXN + row[1:2, :]
    ebase = 2 * (pk // 2)
    eq_l = (plr == ebase).astype(F32)
    eq_r = (plr == ebase + 1).astype(F32)
    v_l = jnp.dot(eq_l, upd, precision=HIGH, preferred_element_type=F32)
    v_r = jnp.dot(eq_r, upd, precision=HIGH, preferred_element_type=F32)
    v_ref[0] = jnp.concatenate([v_l, v_r], axis=1)


def _sc_scatter_body(hs_ref, c_hbm, v_hbm, idx_v, v_v, g_v, sem):
    core = lax.axis_index("c")
    sub = lax.axis_index("s")
    b = core * 8 + sub // 2
    half = lax.rem(sub, 2)

    pltpu.sync_copy(c_hbm.at[pl.ds(b * 4 + half * 2, 2)], idx_v)
    pltpu.sync_copy(v_hbm.at[pl.ds(b * NUPD + half * 256, 256)], v_v)

    cp0 = pltpu.async_copy(hs_ref.at[idx_v.at[0]], g_v.at[pl.ds(0, 128)], sem)
    cp1 = pltpu.async_copy(hs_ref.at[idx_v.at[1]],
                           g_v.at[pl.ds(128, 128)], sem)
    cp0.wait()
    cp1.wait()

    def add_row(k, _):
        for q in range(8):
            qs = pl.ds(q * 16, 16)
            g_v[k, qs] = g_v[k, qs] + v_v[k, qs]
        return 0

    lax.fori_loop(0, 256, add_row, 0)

    plsc.subcore_barrier()

    cp0 = pltpu.async_copy(g_v.at[pl.ds(0, 128)], hs_ref.at[idx_v.at[0]], sem)
    cp1 = pltpu.async_copy(g_v.at[pl.ds(128, 128)],
                           hs_ref.at[idx_v.at[1]], sem)
    cp0.wait()
    cp1.wait()


def _dense_phase(bond_index, bond_feat_cate, bond_feat_float, bond_mask,
                 structure_feat_cate, structure_feat_float,
                 bond_cate_table, bond_cate_ln_g, bond_cate_ln_b,
                 bond_float_W, bond_float_b, bond_float_ln_g, bond_float_ln_b,
                 struct_cate_table, struct_cate_ln_g, struct_cate_ln_b,
                 struct_float_W, struct_float_b, struct_float_ln_g,
                 struct_float_ln_b, virtual_edge_emb, rev_W, rev_b):
    sfc = jnp.pad(structure_feat_cate,
                  ((0, 0), (1, 0), (1, 0), (0, 0))).reshape(B, PROWS, 2, 2)
    sff = jnp.pad(structure_feat_float,
                  ((0, 0), (1, 0), (1, 0), (0, 0))).reshape(B, PROWS, 2, 4)
    sfc_e, sfc_o = sfc[:, :, 0, :], sfc[:, :, 1, :]
    sff_e, sff_o = sff[:, :, 0, :], sff[:, :, 1, :]
    bmask = bond_mask.reshape(B, E, 1)
    bidx_a = bond_index.reshape(B, 2, 2, 128)
    fwd = jnp.stack([bond_index[:, 0, :], bond_index[:, 1, :]], axis=-1)
    bwd = jnp.stack([bond_index[:, 1, :], bond_index[:, 0, :]], axis=-1)
    bidx_col = jnp.concatenate([fwd, bwd], axis=1)
    bidx_row = jnp.transpose(bidx_col, (0, 2, 1))

    row2 = lambda x: x.reshape(1, H)
    weights = (
        bond_cate_table, row2(bond_cate_ln_g), row2(bond_cate_ln_b),
        bond_float_W, row2(bond_float_b), row2(bond_float_ln_g),
        row2(bond_float_ln_b),
        struct_cate_table, row2(struct_cate_ln_g), row2(struct_cate_ln_b),
        struct_float_W, row2(struct_float_b), row2(struct_float_ln_g),
        row2(struct_float_ln_b),
        virtual_edge_emb.reshape(1, H), rev_W, row2(rev_b),
    )

    batch_spec = lambda shape: pl.BlockSpec(
        (1,) + shape, lambda b: (b,) + (0,) * len(shape))
    full_spec = lambda arr: pl.BlockSpec(
        arr.shape, lambda b, _r=len(arr.shape): (0,) * _r)

    hs, v_upd, c_idx = pl.pallas_call(
        _dense_body,
        grid=(B,),
        in_specs=[
            batch_spec((PROWS, 2)), batch_spec((PROWS, 2)),
            batch_spec((PROWS, 4)), batch_spec((PROWS, 4)),
            batch_spec((E, 3)), batch_spec((E, 4)), batch_spec((E, 1)),
            batch_spec((2, 2, 128)), batch_spec((NUPD, 2)),
            batch_spec((2, NUPD)),
        ] + [full_spec(w) for w in weights],
        out_specs=[
            batch_spec((PROWS, 2 * H)),
            batch_spec((NUPD, 2 * H)),
            batch_spec((4, 128)),
        ],
        out_shape=[
            jax.ShapeDtypeStruct((B, PROWS, 2 * H), F32),
            jax.ShapeDtypeStruct((B, NUPD, 2 * H), F32),
            jax.ShapeDtypeStruct((B, 4, 128), jnp.int32),
        ],
        compiler_params=pltpu.CompilerParams(
            vmem_limit_bytes=100 * 1024 * 1024,
        ),
    )(sfc_e, sfc_o, sff_e, sff_o, bond_feat_cate, bond_feat_float, bmask,
      bidx_a, bidx_col, bidx_row, *weights)
    return hs, v_upd, c_idx


def kernel(*args):
    hs, v_upd, c_idx = _dense_phase(*args)

    mesh = plsc.VectorSubcoreMesh(core_axis_name="c", subcore_axis_name="s")
    sc_scatter = pl.kernel(
        _sc_scatter_body,
        out_type=(),
        mesh=mesh,
        scratch_types=[
            pltpu.VMEM((2, 128), jnp.int32),
            pltpu.VMEM((256, 2 * H), F32),
            pltpu.VMEM((256, 2 * H), F32),
            pltpu.SemaphoreType.DMA,
        ],
    )
    hs_ref = jax.new_ref(hs.reshape(B * PROWS, 2 * H))
    sc_scatter(hs_ref, c_idx.reshape(B * 4, 128),
               v_upd.reshape(B * NUPD, 2 * H))
    return hs_ref[...].reshape(B, MAXN, MAXN, H)
